# baseline (speedup 1.0000x reference)
import jax
import jax.numpy as jnp
from jax import lax
from jax.experimental import pallas as pl
from jax.experimental.pallas import tpu as pltpu
from jax.experimental.pallas import tpu_sc as plsc

_B = 16384
_D = 64
_L = 16
_NC = 2
_NS = 16
_NW = _NC * _NS
_BPW = _B // _NW
_CH = 128
_NCH = _BPW // _CH


def _bpr_body(uidx_hbm, iidx_hbm, uemb_hbm, iemb_hbm, out_hbm,
              uidx_v, iidx_v, urows_v, irows_v, out_v, sem):
    wid = lax.axis_index("s") * _NC + lax.axis_index("c")
    base = wid * _BPW

    pltpu.sync_copy(uidx_hbm.at[wid], uidx_v)
    pltpu.sync_copy(iidx_hbm.at[wid], iidx_v)

    lane = lax.iota(jnp.int32, _L)
    gat_dnums = lax.GatherDimensionNumbers(
        offset_dims=(), collapsed_slice_dims=(0,), start_index_map=(0,))
    rot_idx = [jnp.bitwise_and(lane + sh, _L - 1) for sh in (8, 4, 2, 1)]

    def _lane_rotate(p, idx):
        return lax.gather(p, idx[:, None], gat_dnums, (1,),
                          mode=lax.GatherScatterMode.PROMISE_IN_BOUNDS)

    def chunk(j, carry):
        cu = pltpu.async_copy(uemb_hbm.at[uidx_v.at[j]], urows_v, sem)
        ci = pltpu.async_copy(iemb_hbm.at[iidx_v.at[j]], irows_v, sem)
        cu.wait()
        ci.wait()
        for g in range(_CH // _L):
            dots = jnp.zeros((_L,), jnp.float32)
            for k in range(_L):
                kk = g * _L + k
                p = (urows_v[kk, pl.ds(0, _L)] * irows_v[kk, pl.ds(0, _L)])
                for c in range(1, _D // _L):
                    p = p + (urows_v[kk, pl.ds(c * _L, _L)]
                             * irows_v[kk, pl.ds(c * _L, _L)])
                for idx in rot_idx:
                    p = p + _lane_rotate(p, idx)
                dots = jnp.where(lane == k, p, dots)
            out_v[pl.ds(j * _CH + g * _L, _L)] = dots
        return carry

    lax.fori_loop(0, _NCH, chunk, 0)
    pltpu.sync_copy(out_v, out_hbm.at[pl.ds(base, _BPW)])


def kernel(users, items, user_emb, item_emb):
    uidx = users.astype(jnp.int32).reshape(_NW, _NCH, _CH)
    iidx = items.astype(jnp.int32).reshape(_NW, _NCH, _CH)
    uemb128 = jnp.pad(user_emb, ((0, 0), (0, 128 - _D)))
    iemb128 = jnp.pad(item_emb, ((0, 0), (0, 128 - _D)))
    mesh = plsc.VectorSubcoreMesh(core_axis_name="c", subcore_axis_name="s")
    run = pl.kernel(
        _bpr_body,
        out_type=jax.ShapeDtypeStruct((_B,), jnp.float32),
        mesh=mesh,
        scratch_types=[
            pltpu.VMEM((_NCH, _CH), jnp.int32),
            pltpu.VMEM((_NCH, _CH), jnp.int32),
            pltpu.VMEM((_CH, 128), jnp.float32),
            pltpu.VMEM((_CH, 128), jnp.float32),
            pltpu.VMEM((_BPW,), jnp.float32),
            pltpu.SemaphoreType.DMA,
        ],
    )
    return run(uidx, iidx, uemb128, iemb128)


# speedup vs baseline: 2.0938x; 2.0938x over previous
import jax
import jax.numpy as jnp
from jax import lax
from jax.experimental import pallas as pl
from jax.experimental.pallas import tpu as pltpu
from jax.experimental.pallas import tpu_sc as plsc

_B = 16384
_D = 64
_L = 16
_NC = 2
_NS = 16
_NW = _NC * _NS
_BPW = _B // _NW
_NG = _BPW // _L
_NBUF = 3
_NBLK = 125000


def _bpr_body(uidx_hbm, iidx_hbm, uemb_hbm, iemb_hbm, out_hbm,
              uidx_v, iidx_v,
              ublk0, iblk0, ublk1, iblk1, ublk2, iblk2,
              out_v, sem0, sem1, sem2):
    wid = lax.axis_index("s") * _NC + lax.axis_index("c")
    base = wid * _BPW
    ublks = (ublk0, ublk1, ublk2)
    iblks = (iblk0, iblk1, iblk2)
    sems = (sem0, sem1, sem2)

    pltpu.sync_copy(uidx_hbm.at[wid], uidx_v)
    pltpu.sync_copy(iidx_hbm.at[wid], iidx_v)

    lane = lax.iota(jnp.int32, _L)
    gat_dnums = lax.GatherDimensionNumbers(
        offset_dims=(), collapsed_slice_dims=(0,), start_index_map=(0,))
    rot_idx = [jnp.bitwise_and(lane + sh, _L - 1) for sh in (8, 4, 2, 1)]

    def _lane_rotate(p, idx):
        return lax.gather(p, idx[:, None], gat_dnums, (1,),
                          mode=lax.GatherScatterMode.PROMISE_IN_BOUNDS)

    def issue(g, b):
        utv = lax.shift_right_logical(uidx_v[pl.ds(g * _L, _L)], 3)
        itv = lax.shift_right_logical(iidx_v[pl.ds(g * _L, _L)], 3)
        for k in range(_L):
            pltpu.async_copy(uemb_hbm.at[utv[k]], ublks[b].at[k], sems[b])
            pltpu.async_copy(iemb_hbm.at[itv[k]], iblks[b].at[k], sems[b])

    def drain(b):
        for k in range(_L):
            pltpu.make_async_copy(uemb_hbm.at[0], ublks[b].at[k], sems[b]).wait()
            pltpu.make_async_copy(iemb_hbm.at[0], iblks[b].at[k], sems[b]).wait()

    def compute(g, b):
        suv = jnp.bitwise_and(uidx_v[pl.ds(g * _L, _L)], 7)
        siv = jnp.bitwise_and(iidx_v[pl.ds(g * _L, _L)], 7)
        ublk = ublks[b]
        iblk = iblks[b]
        dots = jnp.zeros((_L,), jnp.float32)
        for k in range(_L):
            su = suv[k]
            si = siv[k]
            p = ublk[k, su, pl.ds(0, _L)] * iblk[k, si, pl.ds(0, _L)]
            for c in range(1, _D // _L):
                p = p + (ublk[k, su, pl.ds(c * _L, _L)]
                         * iblk[k, si, pl.ds(c * _L, _L)])
            for idx in rot_idx:
                p = p + _lane_rotate(p, idx)
            dots = jnp.where(lane == k, p, dots)
        out_v[pl.ds(g * _L, _L)] = dots

    for b in range(_NBUF - 1):
        issue(b, b)

    def body(jj, carry):
        g = jj * _NBUF
        for b in range(_NBUF):
            @pl.when(g + b + _NBUF - 1 < _NG)
            def _():
                issue(g + b + _NBUF - 1, (b + _NBUF - 1) % _NBUF)

            drain(b)
            compute(g + b, b)
        return carry

    lax.fori_loop(0, _NG // _NBUF, body, 0)
    for g in range(_NG - _NG % _NBUF, _NG):
        b = g % _NBUF
        drain(b)
        compute(g, b)
    pltpu.sync_copy(out_v, out_hbm.at[pl.ds(base, _BPW)])


def kernel(users, items, user_emb, item_emb):
    uidx = users.astype(jnp.int32).reshape(_NW, _BPW)
    iidx = items.astype(jnp.int32).reshape(_NW, _BPW)
    uemb3 = user_emb.reshape(_NBLK, 8, _D)
    iemb3 = item_emb.reshape(_NBLK, 8, _D)
    mesh = plsc.VectorSubcoreMesh(core_axis_name="c", subcore_axis_name="s")
    run = pl.kernel(
        _bpr_body,
        out_type=jax.ShapeDtypeStruct((_B,), jnp.float32),
        mesh=mesh,
        scratch_types=[
            pltpu.VMEM((_BPW,), jnp.int32),
            pltpu.VMEM((_BPW,), jnp.int32),
            pltpu.VMEM((_L, 8, _D), jnp.float32),
            pltpu.VMEM((_L, 8, _D), jnp.float32),
            pltpu.VMEM((_L, 8, _D), jnp.float32),
            pltpu.VMEM((_L, 8, _D), jnp.float32),
            pltpu.VMEM((_L, 8, _D), jnp.float32),
            pltpu.VMEM((_L, 8, _D), jnp.float32),
            pltpu.VMEM((_BPW,), jnp.float32),
            pltpu.SemaphoreType.DMA,
            pltpu.SemaphoreType.DMA,
            pltpu.SemaphoreType.DMA,
        ],
    )
    return run(uidx, iidx, uemb3, iemb3)
